# baseline (speedup 1.0000x reference)
import functools

import jax
import jax.numpy as jnp
from jax import lax
from jax.experimental import pallas as pl
from jax.experimental.pallas import tpu as pltpu

_K = 64
_BISECT_ITERS = 24


def _topksae_kernel(x_ref, we_ref, be_ref, wd_ref, bd_ref,
                    sparse_ref, recon_ref, pre_ref, rmax_ref,
                    lo_ref, hi_ref, tau_ref,
                    *, k, n_rb, n_chunks, c_blk, iters_per_step):
    r = pl.program_id(0)
    d = pl.program_id(1)
    pe = lax.rem(r, 3)
    pb = lax.rem(r + 2, 3)
    pm = lax.rem(r + 1, 3)
    rows = pre_ref.shape[1]

    @pl.when(r < n_rb)
    def _encode():
        xc = x_ref[...] - bd_ref[...]
        pre = jnp.dot(xc, we_ref[...], preferred_element_type=jnp.float32)
        pre = jnp.maximum(pre + be_ref[...], 0.0)
        pre_ref[pe, :, pl.ds(d * c_blk, c_blk)] = pre
        cmax = jnp.max(pre, axis=1, keepdims=True)

        @pl.when(d == 0)
        def _():
            rmax_ref[pe] = cmax

        @pl.when(d != 0)
        def _():
            rmax_ref[pe] = jnp.maximum(rmax_ref[pe], cmax)

    @pl.when(jnp.logical_and(r >= 1, r <= n_rb))
    def _bisect():
        @pl.when(d == 0)
        def _():
            lo_ref[pb] = jnp.zeros((rows, 1), jnp.int32)
            hi_ref[pb] = lax.bitcast_convert_type(rmax_ref[pb],
                                                  jnp.int32) + 1
            tau_ref[pb] = jnp.full((rows, 1), -1, jnp.int32)

        lo = lo_ref[pb]
        hi = hi_ref[pb]
        tau = tau_ref[pb]
        for _ in range(iters_per_step):
            mid = lo + (hi - lo) // 2
            midf = lax.bitcast_convert_type(mid, jnp.float32)
            cnt = jnp.zeros((rows, 1), jnp.int32)
            for c in range(n_chunks):
                cnt = cnt + jnp.sum(
                    (pre_ref[pb, :, pl.ds(c * c_blk, c_blk)]
                     >= midf).astype(jnp.int32), axis=1, keepdims=True)
            ge = cnt >= k
            tau = jnp.where(jnp.logical_and(tau < 0, cnt == k), mid, tau)
            lo = jnp.where(ge, mid, lo)
            hi = jnp.where(ge, hi, mid)
        lo_ref[pb] = lo
        hi_ref[pb] = hi

        @pl.when(d == n_chunks - 1)
        def _():
            tau_ref[pb] = jnp.where(tau < 0, lo, tau)

        @pl.when(d != n_chunks - 1)
        def _():
            tau_ref[pb] = tau

    @pl.when(r >= 2)
    def _mask_decode():
        chunk = pre_ref[pm, :, pl.ds(d * c_blk, c_blk)]
        tauf = lax.bitcast_convert_type(tau_ref[pm], jnp.float32)
        masked = jnp.where(chunk >= tauf, chunk, 0.0)
        sparse_ref[...] = masked
        part = jnp.dot(masked.astype(jnp.bfloat16), wd_ref[...],
                       preferred_element_type=jnp.float32)

        @pl.when(d == 0)
        def _():
            recon_ref[...] = part + bd_ref[...]

        @pl.when(d != 0)
        def _():
            recon_ref[...] = recon_ref[...] + part


def _run(x, W_enc, b_enc, W_dec, b_dec, *, k, r_blk, n_chunks):
    n_tok, d_in = x.shape
    d_sae = W_enc.shape[1]
    c_blk = d_sae // n_chunks
    n_rb = n_tok // r_blk
    assert _BISECT_ITERS % n_chunks == 0

    grid = (n_rb + 2, n_chunks)
    kern = functools.partial(_topksae_kernel, k=k, n_rb=n_rb,
                             n_chunks=n_chunks, c_blk=c_blk,
                             iters_per_step=_BISECT_ITERS // n_chunks)

    def enc_row(r, d):
        return jnp.minimum(r, n_rb - 1)

    def out_row(r, d):
        return jnp.maximum(r - 2, 0)

    sparse, recon = pl.pallas_call(
        kern,
        grid=grid,
        in_specs=[
            pl.BlockSpec((r_blk, d_in), lambda r, d: (enc_row(r, d), 0)),
            pl.BlockSpec((d_in, c_blk), lambda r, d: (0, d)),
            pl.BlockSpec((1, c_blk), lambda r, d: (0, d)),
            pl.BlockSpec((c_blk, d_in), lambda r, d: (d, 0)),
            pl.BlockSpec((1, d_in), lambda r, d: (0, 0)),
        ],
        out_specs=[
            pl.BlockSpec((r_blk, c_blk), lambda r, d: (out_row(r, d), d)),
            pl.BlockSpec((r_blk, d_in), lambda r, d: (out_row(r, d), 0)),
        ],
        out_shape=[
            jax.ShapeDtypeStruct((n_tok, d_sae), jnp.float32),
            jax.ShapeDtypeStruct((n_tok, d_in), jnp.float32),
        ],
        scratch_shapes=[
            pltpu.VMEM((3, r_blk, d_sae), jnp.float32),
            pltpu.VMEM((3, r_blk, 1), jnp.float32),
            pltpu.VMEM((3, r_blk, 1), jnp.int32),
            pltpu.VMEM((3, r_blk, 1), jnp.int32),
            pltpu.VMEM((3, r_blk, 1), jnp.int32),
        ],
        compiler_params=pltpu.CompilerParams(
            dimension_semantics=("arbitrary", "arbitrary"),
        ),
    )(x, W_enc, b_enc.reshape(1, -1), W_dec.astype(jnp.bfloat16),
      b_dec.reshape(1, -1))
    return recon, sparse


def kernel(x, W_enc, b_enc, W_dec, b_dec):
    return _run(x, W_enc, b_enc, W_dec, b_dec, k=_K, r_blk=256, n_chunks=12)
